# Initial kernel scaffold; baseline (speedup 1.0000x reference)
import jax
import jax.numpy as jnp
from jax.experimental import pallas as pl


def kernel(feat, edge_index, W1, b1, W_mu, b_mu, W_ls, b_ls, noise):
    raise NotImplementedError("write your pallas kernel here")


# speedup vs baseline: 4.6247x; 4.6247x over previous
import functools

import jax
import jax.numpy as jnp
from jax import lax
from jax.experimental import pallas as pl
from jax.experimental.pallas import tpu as pltpu
from jax.experimental.pallas import tpu_sc as plsc

NC = 2
NS = 16
NW = NC * NS

D = 128
EB = 128


def _make_deg_kernel(NP, CH2):
    rpt = NP // NS

    mesh = plsc.VectorSubcoreMesh(core_axis_name="c", subcore_axis_name="s",
                                  num_cores=NC, num_subcores=NS)

    @functools.partial(
        pl.kernel,
        out_type=jax.ShapeDtypeStruct((NC, NP, D), jnp.float32),
        mesh=mesh,
        scratch_types=[
            pltpu.VMEM((CH2, EB), jnp.int32),
            pltpu.VMEM((EB, D), jnp.float32),
            pltpu.VMEM_SHARED((NP, D), jnp.float32),
        ],
    )
    def deg_k(idxm, out, idx_v, ones_v, acc_sh):
        c = lax.axis_index("c")
        s = lax.axis_index("s")
        row0 = s * rpt

        def z_body(i, _):
            ones_v[i // 8, pl.ds((i % 8) * 16, 16)] = jnp.zeros((16,), jnp.float32)
            return _
        lax.fori_loop(0, EB * 8, z_body, None)
        for k in range(rpt // EB):
            pltpu.sync_copy(ones_v, acc_sh.at[pl.ds(row0 + k * EB, EB)])

        def o_body(i, _):
            ones_v[i // 8, pl.ds((i % 8) * 16, 16)] = jnp.ones((16,), jnp.float32)
            return _
        lax.fori_loop(0, EB * 8, o_body, None)
        plsc.subcore_barrier()

        pltpu.sync_copy(idxm.at[c, s], idx_v)

        def body(j, _):
            pltpu.sync_copy(ones_v, acc_sh.at[idx_v.at[j]], add=True)
            return _
        lax.fori_loop(0, CH2, body, None)
        plsc.subcore_barrier()

        pltpu.sync_copy(acc_sh.at[pl.ds(row0, rpt)],
                        out.at[c, pl.ds(row0, rpt)])

    return deg_k


def _make_edge_pass(NP, CH):
    rpt = NP // NS

    mesh = plsc.VectorSubcoreMesh(core_axis_name="c", subcore_axis_name="s",
                                  num_cores=NC, num_subcores=NS)

    @functools.partial(
        pl.kernel,
        out_type=jax.ShapeDtypeStruct((NC, NP, D), jnp.float32),
        mesh=mesh,
        scratch_types=[
            pltpu.VMEM((CH, EB), jnp.int32),
            pltpu.VMEM((CH, EB), jnp.int32),
            pltpu.VMEM((EB, D), jnp.float32),
            pltpu.VMEM_SHARED((NP, D), jnp.float32),
        ],
    )
    def pass_k(table, srcm, dstm, out, src_v, dst_v, rows_v, acc_sh):
        c = lax.axis_index("c")
        s = lax.axis_index("s")
        w = c * NS + s
        row0 = s * rpt

        def z_body(i, _):
            rows_v[i // 8, pl.ds((i % 8) * 16, 16)] = jnp.zeros((16,), jnp.float32)
            return _
        lax.fori_loop(0, EB * 8, z_body, None)
        for k in range(rpt // EB):
            pltpu.sync_copy(rows_v, acc_sh.at[pl.ds(row0 + k * EB, EB)])
        plsc.subcore_barrier()

        pltpu.sync_copy(srcm.at[w], src_v)
        pltpu.sync_copy(dstm.at[w], dst_v)

        def body(j, _):
            pltpu.sync_copy(table.at[src_v.at[j]], rows_v)
            pltpu.sync_copy(rows_v, acc_sh.at[dst_v.at[j]], add=True)
            return _
        lax.fori_loop(0, CH, body, None)
        plsc.subcore_barrier()

        pltpu.sync_copy(acc_sh.at[pl.ds(row0, rpt)],
                        out.at[c, pl.ds(row0, rpt)])

    return pass_k


def _tc1(degparts, feat, NP, BR):
    grid = (NP // BR,)

    def body(dp_ref, feat_ref, t1_ref, no_ref, ni_ref):
        dego = dp_ref[0][:, 0:16]
        degi = dp_ref[1][:, 0:16]
        no = lax.rsqrt(jnp.clip(dego, 1.0, None))
        ni = lax.rsqrt(jnp.clip(degi, 1.0, None))
        no_ref[...] = no
        ni_ref[...] = ni
        t1_ref[...] = feat_ref[...] * no[:, 0:1]

    N = feat.shape[0]
    return pl.pallas_call(
        body,
        grid=grid,
        in_specs=[
            pl.BlockSpec((NC, BR, D), lambda i: (0, i, 0)),
            pl.BlockSpec((BR, D), lambda i: (i, 0)),
        ],
        out_specs=[
            pl.BlockSpec((BR, D), lambda i: (i, 0)),
            pl.BlockSpec((BR, 16), lambda i: (i, 0)),
            pl.BlockSpec((BR, 16), lambda i: (i, 0)),
        ],
        out_shape=[
            jax.ShapeDtypeStruct((NP, D), jnp.float32),
            jax.ShapeDtypeStruct((NP, 16), jnp.float32),
            jax.ShapeDtypeStruct((NP, 16), jnp.float32),
        ],
    )(degparts, feat)


def _tc2(p1, ni16, W1, b1, no16, NP, BR):
    grid = (NP // BR,)

    def body(p_ref, ni_ref, w_ref, b_ref, no_ref, out_ref):
        agg = (p_ref[0] + p_ref[1]) * ni_ref[...][:, 0:1]
        h = jnp.dot(agg, w_ref[...], preferred_element_type=jnp.float32)
        h = jnp.maximum(h + b_ref[...], 0.0)
        out_ref[...] = h * no_ref[...][:, 0:1]

    return pl.pallas_call(
        body,
        grid=grid,
        in_specs=[
            pl.BlockSpec((NC, BR, D), lambda i: (0, i, 0)),
            pl.BlockSpec((BR, 16), lambda i: (i, 0)),
            pl.BlockSpec((D, D), lambda i: (0, 0)),
            pl.BlockSpec((1, D), lambda i: (0, 0)),
            pl.BlockSpec((BR, 16), lambda i: (i, 0)),
        ],
        out_specs=pl.BlockSpec((BR, D), lambda i: (i, 0)),
        out_shape=jax.ShapeDtypeStruct((NP, D), jnp.float32),
    )(p1, ni16, W1, b1, no16)


def _tc3(p2, ni16, W_mu, b_mu, W_ls, b_ls, noise, N, BR):
    grid = (N // BR,)

    def body(p_ref, ni_ref, wm_ref, bm_ref, wl_ref, bl_ref, nz_ref, out_ref):
        g = (p_ref[0] + p_ref[1]) * ni_ref[...][:, 0:1]
        mu = jnp.dot(g, wm_ref[...], preferred_element_type=jnp.float32) + bm_ref[...]
        ls = jnp.dot(g, wl_ref[...], preferred_element_type=jnp.float32) + bl_ref[...]
        out_ref[...] = mu + nz_ref[...] * jnp.exp(ls)

    return pl.pallas_call(
        body,
        grid=grid,
        in_specs=[
            pl.BlockSpec((NC, BR, D), lambda i: (0, i, 0)),
            pl.BlockSpec((BR, 16), lambda i: (i, 0)),
            pl.BlockSpec((D, D), lambda i: (0, 0)),
            pl.BlockSpec((1, D), lambda i: (0, 0)),
            pl.BlockSpec((D, D), lambda i: (0, 0)),
            pl.BlockSpec((1, D), lambda i: (0, 0)),
            pl.BlockSpec((BR, D), lambda i: (i, 0)),
        ],
        out_specs=pl.BlockSpec((BR, D), lambda i: (i, 0)),
        out_shape=jax.ShapeDtypeStruct((N, D), jnp.float32),
    )(p2, ni16, W_mu, b_mu, W_ls, b_ls, noise)


def kernel(feat, edge_index, W1, b1, W_mu, b_mu, W_ls, b_ls, noise):
    N = feat.shape[0]
    E = edge_index.shape[1]

    NP = -(-(N + 1) // (NS * EB)) * (NS * EB)
    CH = -(-E // (NW * EB))
    E_pad = NW * CH * EB

    pad = E_pad - E
    fill = jnp.full((pad,), N, dtype=jnp.int32)
    srcm = jnp.concatenate([edge_index[0], fill]).reshape(NW, CH, EB)
    dstm = jnp.concatenate([edge_index[1], fill]).reshape(NW, CH, EB)

    CH2 = -(-E // (NS * EB))
    E_pad2 = NS * CH2 * EB
    fill2 = jnp.full((E_pad2 - E,), N, dtype=jnp.int32)
    idxm = jnp.stack([
        jnp.concatenate([edge_index[0], fill2]).reshape(NS, CH2, EB),
        jnp.concatenate([edge_index[1], fill2]).reshape(NS, CH2, EB),
    ])

    deg_k = _make_deg_kernel(NP, CH2)
    pass_k = _make_edge_pass(NP, CH)

    degparts = deg_k(idxm)
    table1, no16, ni16 = _tc1(degparts, feat, NP, 1024)
    p1 = pass_k(table1, srcm, dstm)
    table2 = _tc2(p1, ni16, W1, b1.reshape(1, D), no16, NP, 1024)
    p2 = pass_k(table2, srcm, dstm)
    z = _tc3(p2, ni16, W_mu, b_mu.reshape(1, D), W_ls, b_ls.reshape(1, D),
             noise, N, 1000)
    return z


# speedup vs baseline: 7.9151x; 1.7115x over previous
import functools

import jax
import jax.numpy as jnp
from jax import lax
from jax.experimental import pallas as pl
from jax.experimental.pallas import tpu as pltpu
from jax.experimental.pallas import tpu_sc as plsc

NC = 2
NS = 16
NW = NC * NS

D = 128
EB = 128
RING = 8


def _make_deg_kernel(NP, CH2):
    rpt = NP // NS

    mesh = plsc.VectorSubcoreMesh(core_axis_name="c", subcore_axis_name="s",
                                  num_cores=NC, num_subcores=NS)

    @functools.partial(
        pl.kernel,
        out_type=jax.ShapeDtypeStruct((NC, NP, D), jnp.float32),
        mesh=mesh,
        scratch_types=[
            pltpu.VMEM((CH2, EB), jnp.int32),
            pltpu.VMEM((EB, D), jnp.float32),
            pltpu.VMEM_SHARED((NP, D), jnp.float32),
            pltpu.SemaphoreType.DMA,
        ],
    )
    def deg_k(idxm, out, idx_v, ones_v, acc_sh, sem):
        c = lax.axis_index("c")
        s = lax.axis_index("s")
        row0 = s * rpt

        def z_body(i, _):
            ones_v[i // 8, pl.ds((i % 8) * 16, 16)] = jnp.zeros((16,), jnp.float32)
            return _
        lax.fori_loop(0, EB * 8, z_body, None)
        for k in range(rpt // EB):
            pltpu.sync_copy(ones_v, acc_sh.at[pl.ds(row0 + k * EB, EB)])

        def o_body(i, _):
            ones_v[i // 8, pl.ds((i % 8) * 16, 16)] = jnp.ones((16,), jnp.float32)
            return _
        lax.fori_loop(0, EB * 8, o_body, None)
        plsc.subcore_barrier()

        pltpu.sync_copy(idxm.at[c, s], idx_v)

        for k in range(RING):
            pltpu.async_copy(ones_v, acc_sh.at[idx_v.at[k]], sem, add=True)

        def body(j, _):
            pltpu.make_async_copy(ones_v, acc_sh.at[idx_v.at[j - RING]], sem).wait()
            pltpu.async_copy(ones_v, acc_sh.at[idx_v.at[j]], sem, add=True)
            return _
        lax.fori_loop(RING, CH2, body, None)
        for k in range(CH2 - RING, CH2):
            pltpu.make_async_copy(ones_v, acc_sh.at[idx_v.at[k]], sem).wait()
        plsc.subcore_barrier()

        pltpu.sync_copy(acc_sh.at[pl.ds(row0, rpt)],
                        out.at[c, pl.ds(row0, rpt)])

    return deg_k


def _make_edge_pass(NP, CH):
    rpt = NP // NS

    mesh = plsc.VectorSubcoreMesh(core_axis_name="c", subcore_axis_name="s",
                                  num_cores=NC, num_subcores=NS)

    NH = CH // 2

    @functools.partial(
        pl.kernel,
        out_type=jax.ShapeDtypeStruct((NC, NP, D), jnp.float32),
        mesh=mesh,
        scratch_types=[
            pltpu.VMEM((NH, EB), jnp.int32),
            pltpu.VMEM((NH, EB), jnp.int32),
            pltpu.VMEM((EB, D), jnp.float32),
            pltpu.VMEM((EB, D), jnp.float32),
            pltpu.VMEM_SHARED((NP, D), jnp.float32),
            pltpu.SemaphoreType.DMA,
            pltpu.SemaphoreType.DMA,
        ],
    )
    def pass_k(table, srcm, dstm, out, src_v, dst_v, buf0, buf1, acc_sh,
               sem0, sem1):
        c = lax.axis_index("c")
        s = lax.axis_index("s")
        w = c * NS + s
        row0 = s * rpt

        def z_body(i, _):
            buf0[i // 8, pl.ds((i % 8) * 16, 16)] = jnp.zeros((16,), jnp.float32)
            return _
        lax.fori_loop(0, EB * 8, z_body, None)
        for k in range(rpt // EB):
            pltpu.sync_copy(buf0, acc_sh.at[pl.ds(row0 + k * EB, EB)])
        plsc.subcore_barrier()

        for h in range(2):
            pltpu.sync_copy(srcm.at[w, pl.ds(h * NH, NH)], src_v)
            pltpu.sync_copy(dstm.at[w, pl.ds(h * NH, NH)], dst_v)
            pltpu.async_copy(table.at[src_v.at[0]], buf0, sem0)

            def body(i, _):
                j = 2 * i
                pltpu.make_async_copy(table.at[src_v.at[j]], buf0, sem0).wait()
                pltpu.async_copy(table.at[src_v.at[j + 1]], buf1, sem1)
                pltpu.sync_copy(buf0, acc_sh.at[dst_v.at[j]], add=True)
                pltpu.make_async_copy(table.at[src_v.at[j + 1]], buf1, sem1).wait()

                @pl.when(j + 2 < NH)
                def _next():
                    pltpu.async_copy(table.at[src_v.at[j + 2]], buf0, sem0)

                pltpu.sync_copy(buf1, acc_sh.at[dst_v.at[j + 1]], add=True)
                return _
            lax.fori_loop(0, NH // 2, body, None)
        plsc.subcore_barrier()

        pltpu.sync_copy(acc_sh.at[pl.ds(row0, rpt)],
                        out.at[c, pl.ds(row0, rpt)])

    return pass_k


def _tc1(degparts, feat, NP, BR):
    grid = (NP // BR,)

    def body(dp_ref, feat_ref, t1_ref, no_ref, ni_ref):
        dego = dp_ref[0][:, 0:16]
        degi = dp_ref[1][:, 0:16]
        no = lax.rsqrt(jnp.clip(dego, 1.0, None))
        ni = lax.rsqrt(jnp.clip(degi, 1.0, None))
        no_ref[...] = no
        ni_ref[...] = ni
        t1_ref[...] = feat_ref[...] * no[:, 0:1]

    N = feat.shape[0]
    return pl.pallas_call(
        body,
        grid=grid,
        in_specs=[
            pl.BlockSpec((NC, BR, D), lambda i: (0, i, 0)),
            pl.BlockSpec((BR, D), lambda i: (i, 0)),
        ],
        out_specs=[
            pl.BlockSpec((BR, D), lambda i: (i, 0)),
            pl.BlockSpec((BR, 16), lambda i: (i, 0)),
            pl.BlockSpec((BR, 16), lambda i: (i, 0)),
        ],
        out_shape=[
            jax.ShapeDtypeStruct((NP, D), jnp.float32),
            jax.ShapeDtypeStruct((NP, 16), jnp.float32),
            jax.ShapeDtypeStruct((NP, 16), jnp.float32),
        ],
    )(degparts, feat)


def _tc2(p1, ni16, W1, b1, no16, NP, BR):
    grid = (NP // BR,)

    def body(p_ref, ni_ref, w_ref, b_ref, no_ref, out_ref):
        agg = (p_ref[0] + p_ref[1]) * ni_ref[...][:, 0:1]
        h = jnp.dot(agg, w_ref[...], preferred_element_type=jnp.float32)
        h = jnp.maximum(h + b_ref[...], 0.0)
        out_ref[...] = h * no_ref[...][:, 0:1]

    return pl.pallas_call(
        body,
        grid=grid,
        in_specs=[
            pl.BlockSpec((NC, BR, D), lambda i: (0, i, 0)),
            pl.BlockSpec((BR, 16), lambda i: (i, 0)),
            pl.BlockSpec((D, D), lambda i: (0, 0)),
            pl.BlockSpec((1, D), lambda i: (0, 0)),
            pl.BlockSpec((BR, 16), lambda i: (i, 0)),
        ],
        out_specs=pl.BlockSpec((BR, D), lambda i: (i, 0)),
        out_shape=jax.ShapeDtypeStruct((NP, D), jnp.float32),
    )(p1, ni16, W1, b1, no16)


def _tc3(p2, ni16, W_mu, b_mu, W_ls, b_ls, noise, N, BR):
    grid = (N // BR,)

    def body(p_ref, ni_ref, wm_ref, bm_ref, wl_ref, bl_ref, nz_ref, out_ref):
        g = (p_ref[0] + p_ref[1]) * ni_ref[...][:, 0:1]
        mu = jnp.dot(g, wm_ref[...], preferred_element_type=jnp.float32) + bm_ref[...]
        ls = jnp.dot(g, wl_ref[...], preferred_element_type=jnp.float32) + bl_ref[...]
        out_ref[...] = mu + nz_ref[...] * jnp.exp(ls)

    return pl.pallas_call(
        body,
        grid=grid,
        in_specs=[
            pl.BlockSpec((NC, BR, D), lambda i: (0, i, 0)),
            pl.BlockSpec((BR, 16), lambda i: (i, 0)),
            pl.BlockSpec((D, D), lambda i: (0, 0)),
            pl.BlockSpec((1, D), lambda i: (0, 0)),
            pl.BlockSpec((D, D), lambda i: (0, 0)),
            pl.BlockSpec((1, D), lambda i: (0, 0)),
            pl.BlockSpec((BR, D), lambda i: (i, 0)),
        ],
        out_specs=pl.BlockSpec((BR, D), lambda i: (i, 0)),
        out_shape=jax.ShapeDtypeStruct((N, D), jnp.float32),
    )(p2, ni16, W_mu, b_mu, W_ls, b_ls, noise)


def kernel(feat, edge_index, W1, b1, W_mu, b_mu, W_ls, b_ls, noise):
    N = feat.shape[0]
    E = edge_index.shape[1]

    NP = -(-(N + 1) // (NS * EB)) * (NS * EB)
    CH = -(-E // (NW * EB))
    CH = -(-CH // 4) * 4
    E_pad = NW * CH * EB

    pad = E_pad - E
    fill = (N + jnp.arange(pad, dtype=jnp.int32) % (NP - N)).astype(jnp.int32)
    srcm = jnp.concatenate([edge_index[0], fill]).reshape(NW, CH, EB)
    dstm = jnp.concatenate([edge_index[1], fill]).reshape(NW, CH, EB)

    CH2 = max(-(-E // (NS * EB)), RING)
    E_pad2 = NS * CH2 * EB
    pad2 = E_pad2 - E
    fill2 = (N + jnp.arange(pad2, dtype=jnp.int32) % (NP - N)).astype(jnp.int32)
    idxm = jnp.stack([
        jnp.concatenate([edge_index[0], fill2]).reshape(NS, CH2, EB),
        jnp.concatenate([edge_index[1], fill2]).reshape(NS, CH2, EB),
    ])

    deg_k = _make_deg_kernel(NP, CH2)
    pass_k = _make_edge_pass(NP, CH)

    degparts = deg_k(idxm)
    table1, no16, ni16 = _tc1(degparts, feat, NP, 1024)
    p1 = pass_k(table1, srcm, dstm)
    table2 = _tc2(p1, ni16, W1, b1.reshape(1, D), no16, NP, 1024)
    p2 = pass_k(table2, srcm, dstm)
    z = _tc3(p2, ni16, W_mu, b_mu.reshape(1, D), W_ls, b_ls.reshape(1, D),
             noise, N, 1000)
    return z
